# baseline (speedup 1.0000x reference)
import functools

import jax
import jax.numpy as jnp
from jax import lax
from jax.experimental import pallas as pl
from jax.experimental.pallas import tpu as pltpu
from jax.experimental.pallas import tpu_sc as plsc

_NC = 2
_NS = 16
_L = 16
_C = 64
_BE = 1024
_K = 12800
_KM = _K - 640
_SENT = 1 << 20


def _spmm_sc(n, d, nblk, b, edges3):
    nw = _NC * _NS
    rpt = -(-n // (nw * 8)) * 8
    npad = rpt * nw

    mesh = plsc.VectorSubcoreMesh(core_axis_name="c", subcore_axis_name="s")

    @functools.partial(
        pl.kernel,
        out_type=jax.ShapeDtypeStruct((npad, d), jnp.float32),
        mesh=mesh,
        compiler_params=pltpu.CompilerParams(needs_layout_passes=False),
        scratch_types=[
            pltpu.VMEM((4, 3, _BE), jnp.int32),
            pltpu.VMEM((_K,), jnp.int32),
            pltpu.VMEM((_K,), jnp.int32),
            pltpu.VMEM((_K,), jnp.float32),
            pltpu.VMEM((4, _C, d), jnp.float32),
            pltpu.VMEM((rpt, d), jnp.float32),
            pltpu.SemaphoreType.DMA,
            pltpu.SemaphoreType.DMA,
            pltpu.SemaphoreType.DMA,
            pltpu.SemaphoreType.DMA,
            pltpu.SemaphoreType.DMA,
            pltpu.SemaphoreType.DMA,
            pltpu.SemaphoreType.DMA,
            pltpu.SemaphoreType.DMA,
        ],
    )
    def spmm(b_hbm, e3_hbm, out_hbm,
             ering, lrows, lcols, lvals, gbuf, acc,
             ss0, ss1, ss2, ss3, sg0, sg1, sg2, sg3):
        ssem = (ss0, ss1, ss2, ss3)
        gsem = (sg0, sg1, sg2, sg3)
        ci = lax.axis_index("c")
        si = lax.axis_index("s")
        wid = ci * _NS + si
        lo = wid * rpt
        iot = lax.iota(jnp.int32, _L)
        zero = jnp.zeros((_L,), jnp.float32)

        def _z(i, carry):
            for k in range(d // _L):
                acc[i, pl.ds(k * _L, _L)] = zero
            return carry
        lax.fori_loop(0, rpt, _z, 0)

        def blk_copy(blk, p):
            return pltpu.make_async_copy(e3_hbm.at[blk], ering.at[p],
                                         ssem[p])

        def scan_block(p, off):
            @plsc.parallel_loop(0, _BE // _L, carry=off, unroll=4)
            def group(g, off):
                sl = pl.ds(g * _L, _L)
                r16 = ering[p, 0, sl]
                loc = r16 - lo
                keep = loc.astype(jnp.uint32) < jnp.uint32(rpt)
                pos = plsc.cumsum(jnp.where(keep, 1, 0))
                addr = jnp.minimum(off + pos - 1, _K - 1)
                plsc.store_scatter(lrows, [addr], loc, mask=keep)
                plsc.store_scatter(lcols, [addr], ering[p, 1, sl], mask=keep)
                plsc.store_scatter(
                    lvals, [addr],
                    plsc.bitcast(ering[p, 2, sl], jnp.float32), mask=keep)
                return off + plsc.all_reduce_population_count(keep)
            return group

        offv = jnp.zeros((_L,), jnp.int32)
        for p in range(4):
            blk_copy(p, p).start()

        def scan_step(t, off):
            b0 = 4 * t
            for p in range(4):
                blk_copy(b0 + p, p).wait()
                off = scan_block(p, off)
                blk_copy(b0 + 4 + p, p).start()
            return off
        offv = lax.fori_loop(0, nblk // 4 - 1, scan_step, offv)
        for p in range(4):
            blk_copy(nblk - 4 + p, p).wait()
            offv = scan_block(p, offv)

        kcount = jnp.int32(0)
        for bit in range(15):
            one = jnp.bitwise_or.reduce((offv >> bit) & 1 != 0)
            kcount = kcount + (one.astype(jnp.int32) << bit)
        kcount = jnp.minimum(kcount, _KM)

        izero = jnp.zeros((_L,), jnp.int32)

        def padg(g, carry):
            addr = jnp.minimum(offv, _KM) + g * _L + iot
            plsc.store_scatter(lrows, [addr], izero)
            plsc.store_scatter(lcols, [addr], izero)
            plsc.store_scatter(lvals, [addr], zero)
            return carry
        lax.fori_loop(0, 512 // _L, padg, 0)

        def gather(j, p):
            return pltpu.make_async_copy(
                b_hbm.at[lcols.at[pl.ds(j * _C, _C)]], gbuf.at[p], gsem[p])

        cks = [k * _L + iot for k in range(d // _L)]

        def process(j, p):
            buf = gbuf.at[p]

            @plsc.parallel_loop(0, _C // _L, unroll=2)
            def group(g):
                base = j * _C + g * _L
                for t in range(_L):
                    it = jnp.full((_L,), base + t, jnp.int32)
                    vv = plsc.load_gather(lvals, [it])
                    rw = plsc.load_gather(lrows, [it])
                    ei = g * _L + t
                    xs = [buf[ei, pl.ds(k * _L, _L)] for k in range(d // _L)]
                    ps = [x * vv for x in xs]
                    for k in range(d // _L):
                        plsc.addupdate_scatter(acc, [rw, cks[k]], ps[k])

        nch = (kcount + 4 * _C - 1) // (4 * _C) * 4
        for p in range(4):
            gather(p, p).start()

        def step(t, carry):
            j0 = 4 * t
            for p in range(4):
                gather(j0 + p, p).wait()
                process(j0 + p, p)
                gather(j0 + 4 + p, p).start()
            return carry
        lax.fori_loop(0, nch // 4, step, 0)
        for p in range(4):
            gather(nch + p, p).wait()

        pltpu.sync_copy(acc, out_hbm.at[pl.ds(lo, rpt)])

    return spmm(b, edges3)


def kernel(indices, values, shape, b):
    n, d = b.shape
    e = values.shape[0]
    assert d == 8 * _L

    nblk = (-(-e // _BE) + 3) // 4 * 4
    ep = nblk * _BE

    row = indices[0].astype(jnp.int32)
    col = indices[1].astype(jnp.int32)

    rowp = jnp.full((ep,), _SENT, jnp.int32).at[:e].set(row)
    colp = jnp.zeros((ep,), jnp.int32).at[:e].set(col)
    valp = jnp.zeros((ep,), jnp.float32).at[:e].set(values)
    edges3 = jnp.stack([rowp.reshape(nblk, _BE), colp.reshape(nblk, _BE),
                        lax.bitcast_convert_type(valp, jnp.int32).reshape(
                            nblk, _BE)], axis=1)

    out = _spmm_sc(n, d, nblk, b, edges3)
    return out[:n]


# speedup vs baseline: 1.3405x; 1.3405x over previous
import functools

import jax
import jax.numpy as jnp
from jax import lax
from jax.experimental import pallas as pl
from jax.experimental.pallas import tpu as pltpu
from jax.experimental.pallas import tpu_sc as plsc

_NC = 2
_NS = 16
_L = 16
_C = 128
_BE = 1024
_K = 12800
_SENT = 1 << 20


def _spmm_sc(n, d, nblk, b, edges3):
    nw = _NC * _NS
    rpt = -(-n // (nw * 8)) * 8
    npad = rpt * nw

    mesh = plsc.VectorSubcoreMesh(core_axis_name="c", subcore_axis_name="s")

    @functools.partial(
        pl.kernel,
        out_type=jax.ShapeDtypeStruct((npad, d), jnp.float32),
        mesh=mesh,
        compiler_params=pltpu.CompilerParams(needs_layout_passes=False),
        scratch_types=[
            pltpu.VMEM((4, 3, _BE), jnp.int32),
            pltpu.VMEM((_K,), jnp.int32),
            pltpu.VMEM((_K,), jnp.int32),
            pltpu.VMEM((_K,), jnp.float32),
            pltpu.VMEM((_C, d), jnp.float32),
            pltpu.VMEM((_C, d), jnp.float32),
            pltpu.VMEM((rpt, d), jnp.float32),
            pltpu.SemaphoreType.DMA,
            pltpu.SemaphoreType.DMA,
            pltpu.SemaphoreType.DMA,
            pltpu.SemaphoreType.DMA,
            pltpu.SemaphoreType.DMA,
            pltpu.SemaphoreType.DMA,
        ],
    )
    def spmm(b_hbm, e3_hbm, out_hbm,
             ering, lrows, lcols, lvals, buf0, buf1, acc,
             semS0, semS1, semS2, semS3, sem0, sem1):
        ssem = (semS0, semS1, semS2, semS3)
        ci = lax.axis_index("c")
        si = lax.axis_index("s")
        wid = ci * _NS + si
        lo = wid * rpt
        iot = lax.iota(jnp.int32, _L)

        zero = jnp.zeros((_L,), jnp.float32)

        def _z(i, carry):
            for k in range(d // _L):
                acc[i, pl.ds(k * _L, _L)] = zero
            return carry
        lax.fori_loop(0, rpt, _z, 0)

        def blk_copy(blk, p, sem):
            return pltpu.make_async_copy(e3_hbm.at[blk], ering.at[p], sem)

        def issue_blk(blk, p, sem):
            blk_copy(blk, p, sem).start()

        def wait_blk(blk, p, sem):
            blk_copy(blk, p, sem).wait()

        dnums = lax.GatherDimensionNumbers(
            offset_dims=(), collapsed_slice_dims=(0,), start_index_map=(0,))

        def bcast(vec, t):
            return lax.gather(vec, jnp.full((_L, 1), t, jnp.int32), dnums,
                              (1,), mode=lax.GatherScatterMode.PROMISE_IN_BOUNDS)

        def shuffle_up(vec, sh):
            idx = jnp.maximum(iot - sh, 0).reshape(_L, 1)
            return lax.gather(vec, idx, dnums, (1,),
                              mode=lax.GatherScatterMode.PROMISE_IN_BOUNDS)

        def prefix_incl(x):
            for sh in (1, 2, 4, 8):
                x = x + jnp.where(iot >= sh, shuffle_up(x, sh), 0)
            return x

        def scan_block(p, off):
            @plsc.parallel_loop(0, _BE // _L, carry=off, unroll=4)
            def group(g, off):
                sl = pl.ds(g * _L, _L)
                r16 = ering[p, 0, sl]
                loc = r16 - lo
                keep = loc.astype(jnp.uint32) < jnp.uint32(rpt)
                pos = plsc.cumsum(jnp.where(keep, 1, 0))
                addr = jnp.minimum(off + pos - 1, _K - 1)
                plsc.store_scatter(lrows, [addr], loc, mask=keep)
                plsc.store_scatter(lcols, [addr], ering[p, 1, sl], mask=keep)
                plsc.store_scatter(
                    lvals, [addr],
                    plsc.bitcast(ering[p, 2, sl], jnp.float32), mask=keep)
                return off + plsc.all_reduce_population_count(keep)
            return group

        offv = jnp.zeros((_L,), jnp.int32)
        for p in range(4):
            issue_blk(p, p, ssem[p])

        def scan_step(t, off):
            b0 = 4 * t
            for p in range(4):
                wait_blk(b0 + p, p, ssem[p])
                off = scan_block(p, off)
                issue_blk(b0 + 4 + p, p, ssem[p])
            return off
        offv = lax.fori_loop(0, nblk // 4 - 1, scan_step, offv)
        for p in range(4):
            wait_blk(nblk - 4 + p, p, ssem[p])
            offv = scan_block(p, offv)

        kcount = jnp.int32(0)
        for bit in range(15):
            one = jnp.bitwise_or.reduce((offv >> bit) & 1 != 0)
            kcount = kcount + (one.astype(jnp.int32) << bit)
        kcount = jnp.minimum(kcount, _K - 5 * _C)

        izero = jnp.zeros((_L,), jnp.int32)

        def padg(g, carry):
            addr = jnp.minimum(offv, _K - 5 * _C) + g * _L + iot
            plsc.store_scatter(lrows, [addr], izero)
            plsc.store_scatter(lcols, [addr], izero)
            plsc.store_scatter(lvals, [addr], zero)
            return carry
        lax.fori_loop(0, 4 * _C // _L, padg, 0)

        def gather(j, buf, sem):
            return pltpu.make_async_copy(
                b_hbm.at[lcols.at[pl.ds(j * _C, _C)]], buf, sem)

        def process(j, buf):
            cks = [k * _L + iot for k in range(d // _L)]

            @plsc.parallel_loop(0, _C // _L, unroll=2)
            def group(g):
                base = j * _C + g * _L
                for t in range(_L):
                    it = jnp.full((_L,), base + t, jnp.int32)
                    vv = plsc.load_gather(lvals, [it])
                    rw = plsc.load_gather(lrows, [it])
                    ei = g * _L + t
                    xs = [buf[ei, pl.ds(k * _L, _L)] for k in range(d // _L)]
                    ps = [x * vv for x in xs]
                    for k in range(d // _L):
                        plsc.addupdate_scatter(acc, [rw, cks[k]], ps[k])

        nch = (kcount + 2 * _C - 1) // (2 * _C) * 2
        gather(0, buf0, sem0).start()

        def step(t, carry):
            j0 = 2 * t
            gather(j0 + 1, buf1, sem1).start()
            gather(j0, buf0, sem0).wait()
            process(j0, buf0)
            gather(j0 + 2, buf0, sem0).start()
            gather(j0 + 1, buf1, sem1).wait()
            process(j0 + 1, buf1)
            return carry
        lax.fori_loop(0, nch // 2, step, 0)
        gather(nch, buf0, sem0).wait()

        pltpu.sync_copy(acc, out_hbm.at[pl.ds(lo, rpt)])

    return spmm(b, edges3)


def kernel(indices, values, shape, b):
    n, d = b.shape
    e = values.shape[0]
    assert d == 8 * _L

    nblk = (-(-e // _BE) + 3) // 4 * 4
    ep = nblk * _BE

    row = indices[0].astype(jnp.int32)
    col = indices[1].astype(jnp.int32)

    rowp = jnp.full((ep,), _SENT, jnp.int32).at[:e].set(row)
    colp = jnp.zeros((ep,), jnp.int32).at[:e].set(col)
    valp = jnp.zeros((ep,), jnp.float32).at[:e].set(values)
    edges3 = jnp.stack([rowp.reshape(nblk, _BE), colp.reshape(nblk, _BE),
                        lax.bitcast_convert_type(valp, jnp.int32).reshape(
                            nblk, _BE)], axis=1)

    out = _spmm_sc(n, d, nblk, b, edges3)
    return out[:n]


# speedup vs baseline: 1.3441x; 1.0027x over previous
import functools

import jax
import jax.numpy as jnp
from jax import lax
from jax.experimental import pallas as pl
from jax.experimental.pallas import tpu as pltpu
from jax.experimental.pallas import tpu_sc as plsc

_NC = 2
_NS = 16
_L = 16
_C = 128
_BE = 1024
_K = 12800
_SENT = 1 << 20


def _spmm_sc(n, d, nblk, b, edges3):
    nw = _NC * _NS
    rpt = -(-n // (nw * 8)) * 8
    npad = rpt * nw

    mesh = plsc.VectorSubcoreMesh(core_axis_name="c", subcore_axis_name="s")

    @functools.partial(
        pl.kernel,
        out_type=jax.ShapeDtypeStruct((npad, d), jnp.float32),
        mesh=mesh,
        compiler_params=pltpu.CompilerParams(needs_layout_passes=False),
        scratch_types=[
            pltpu.VMEM((4, 3, _BE), jnp.int32),
            pltpu.VMEM((_K,), jnp.int32),
            pltpu.VMEM((_K,), jnp.int32),
            pltpu.VMEM((_K,), jnp.float32),
            pltpu.VMEM((_C, d), jnp.float32),
            pltpu.VMEM((_C, d), jnp.float32),
            pltpu.VMEM((rpt, d), jnp.float32),
            pltpu.SemaphoreType.DMA,
            pltpu.SemaphoreType.DMA,
            pltpu.SemaphoreType.DMA,
            pltpu.SemaphoreType.DMA,
            pltpu.SemaphoreType.DMA,
            pltpu.SemaphoreType.DMA,
        ],
    )
    def spmm(b_hbm, e3_hbm, out_hbm,
             ering, lrows, lcols, lvals, buf0, buf1, acc,
             semS0, semS1, semS2, semS3, sem0, sem1):
        ssem = (semS0, semS1, semS2, semS3)
        ci = lax.axis_index("c")
        si = lax.axis_index("s")
        wid = ci * _NS + si
        lo = wid * rpt
        iot = lax.iota(jnp.int32, _L)

        zero = jnp.zeros((_L,), jnp.float32)

        def _z(i, carry):
            for k in range(d // _L):
                acc[i, pl.ds(k * _L, _L)] = zero
            return carry
        lax.fori_loop(0, rpt, _z, 0)

        def blk_copy(blk, p, sem):
            return pltpu.make_async_copy(e3_hbm.at[blk], ering.at[p], sem)

        def issue_blk(blk, p, sem):
            blk_copy(blk, p, sem).start()

        def wait_blk(blk, p, sem):
            blk_copy(blk, p, sem).wait()

        dnums = lax.GatherDimensionNumbers(
            offset_dims=(), collapsed_slice_dims=(0,), start_index_map=(0,))

        def bcast(vec, t):
            return lax.gather(vec, jnp.full((_L, 1), t, jnp.int32), dnums,
                              (1,), mode=lax.GatherScatterMode.PROMISE_IN_BOUNDS)

        def shuffle_up(vec, sh):
            idx = jnp.maximum(iot - sh, 0).reshape(_L, 1)
            return lax.gather(vec, idx, dnums, (1,),
                              mode=lax.GatherScatterMode.PROMISE_IN_BOUNDS)

        def prefix_incl(x):
            for sh in (1, 2, 4, 8):
                x = x + jnp.where(iot >= sh, shuffle_up(x, sh), 0)
            return x

        def scan_block(p, off):
            @plsc.parallel_loop(0, _BE // _L, carry=off, unroll=8)
            def group(g, off):
                sl = pl.ds(g * _L, _L)
                r16 = ering[p, 0, sl]
                loc = r16 - lo
                keep = loc.astype(jnp.uint32) < jnp.uint32(rpt)
                pos = plsc.cumsum(jnp.where(keep, 1, 0))
                addr = jnp.minimum(off + pos - 1, _K - 1)
                plsc.store_scatter(lrows, [addr], loc, mask=keep)
                plsc.store_scatter(lcols, [addr], ering[p, 1, sl], mask=keep)
                plsc.store_scatter(
                    lvals, [addr],
                    plsc.bitcast(ering[p, 2, sl], jnp.float32), mask=keep)
                return off + plsc.all_reduce_population_count(keep)
            return group

        offv = jnp.zeros((_L,), jnp.int32)
        for p in range(4):
            issue_blk(p, p, ssem[p])

        def scan_step(t, off):
            b0 = 4 * t
            for p in range(4):
                wait_blk(b0 + p, p, ssem[p])
                off = scan_block(p, off)
                issue_blk(b0 + 4 + p, p, ssem[p])
            return off
        offv = lax.fori_loop(0, nblk // 4 - 1, scan_step, offv)
        for p in range(4):
            wait_blk(nblk - 4 + p, p, ssem[p])
            offv = scan_block(p, offv)

        kcount = jnp.int32(0)
        for bit in range(15):
            one = jnp.bitwise_or.reduce((offv >> bit) & 1 != 0)
            kcount = kcount + (one.astype(jnp.int32) << bit)
        kcount = jnp.minimum(kcount, _K - 5 * _C)

        izero = jnp.zeros((_L,), jnp.int32)

        def padg(g, carry):
            addr = jnp.minimum(offv, _K - 5 * _C) + g * _L + iot
            plsc.store_scatter(lrows, [addr], izero)
            plsc.store_scatter(lcols, [addr], izero)
            plsc.store_scatter(lvals, [addr], zero)
            return carry
        lax.fori_loop(0, 4 * _C // _L, padg, 0)

        def gather(j, buf, sem):
            return pltpu.make_async_copy(
                b_hbm.at[lcols.at[pl.ds(j * _C, _C)]], buf, sem)

        def process(j, buf):
            cks = [k * _L + iot for k in range(d // _L)]

            @plsc.parallel_loop(0, _C // _L, unroll=2)
            def group(g):
                base = j * _C + g * _L
                for t in range(_L):
                    it = jnp.full((_L,), base + t, jnp.int32)
                    vv = plsc.load_gather(lvals, [it])
                    rw = plsc.load_gather(lrows, [it])
                    ei = g * _L + t
                    xs = [buf[ei, pl.ds(k * _L, _L)] for k in range(d // _L)]
                    ps = [x * vv for x in xs]
                    for k in range(d // _L):
                        plsc.addupdate_scatter(acc, [rw, cks[k]], ps[k])

        nch = (kcount + 2 * _C - 1) // (2 * _C) * 2
        gather(0, buf0, sem0).start()

        def step(t, carry):
            j0 = 2 * t
            gather(j0 + 1, buf1, sem1).start()
            gather(j0, buf0, sem0).wait()
            process(j0, buf0)
            gather(j0 + 2, buf0, sem0).start()
            gather(j0 + 1, buf1, sem1).wait()
            process(j0 + 1, buf1)
            return carry
        lax.fori_loop(0, nch // 2, step, 0)
        gather(nch, buf0, sem0).wait()

        pltpu.sync_copy(acc, out_hbm.at[pl.ds(lo, rpt)])

    return spmm(b, edges3)


def kernel(indices, values, shape, b):
    n, d = b.shape
    e = values.shape[0]
    assert d == 8 * _L

    nblk = (-(-e // _BE) + 3) // 4 * 4
    ep = nblk * _BE

    row = indices[0].astype(jnp.int32)
    col = indices[1].astype(jnp.int32)

    rowp = jnp.full((ep,), _SENT, jnp.int32).at[:e].set(row)
    colp = jnp.zeros((ep,), jnp.int32).at[:e].set(col)
    valp = jnp.zeros((ep,), jnp.float32).at[:e].set(values)
    edges3 = jnp.stack([rowp.reshape(nblk, _BE), colp.reshape(nblk, _BE),
                        lax.bitcast_convert_type(valp, jnp.int32).reshape(
                            nblk, _BE)], axis=1)

    out = _spmm_sc(n, d, nblk, b, edges3)
    return out[:n]
